# Initial kernel scaffold; baseline (speedup 1.0000x reference)
import jax
import jax.numpy as jnp
from jax.experimental import pallas as pl


def kernel(gesture, audio_embeddings, text_embeddings, vid_indices, cb, cbsq, text_w, text_b, enc_w0, enc_b0, enc_w1, enc_b1, enc_w2, enc_b2, enc_w3, enc_b3, enc_w4, enc_b4, enc_w5, enc_b5, enc_w6, enc_b6, enc_w7, enc_b7, enc_w8, enc_b8, enc_w9, enc_b9, enc_w10, enc_b10, enc_w11, enc_b11, dec_w0, dec_b0, dec_w1, dec_b1, dec_w2, dec_b2, dec_w3, dec_b3, dec_w4, dec_b4, dec_w5, dec_b5, dec_w6, dec_b6, dec_w7, dec_b7, dec_w8, dec_b8, dec_w9, dec_b9, dec_w10, dec_b10, dec_w11, dec_b11, dec_w12, dec_b12):
    raise NotImplementedError("write your pallas kernel here")


# speedup vs baseline: 1.6519x; 1.6519x over previous
import jax
import jax.numpy as jnp
from jax import lax
from jax.experimental import pallas as pl
from jax.experimental.pallas import tpu as pltpu


_VMEM_LIMIT = 48 * 1024 * 1024
_TPAD = 512
_NS = 2

_ENC_CONVS = (
    (((1, -1), (0, 0), (1, 1)), True, 512),
    (((2, -1), (2, 0), (2, 1), (2, 2)), False, 256),
    (((1, -1), (0, 0), (1, 1)), True, 256),
    (((0, 0),), False, 256),
    (((1, -3), (0, 0), (1, 3)), True, 256),
    (((0, 0),), False, 256),
    (((2, -1), (2, 0), (2, 1), (2, 2)), False, 128),
    (((1, -1), (0, 0), (1, 1)), True, 128),
    (((0, 0),), False, 128),
    (((1, -3), (0, 0), (1, 3)), True, 128),
    (((0, 0),), False, 128),
    (((1, -1), (0, 0), (1, 1)), False, 128),
)
_ENC_RES = ((2, 3), (4, 5), (7, 8), (9, 10))


def _enc_vq_kernel(*refs):
    x_ref = refs[0]
    wr = refs[1:25]
    cb_ref, cbsq_ref = refs[25:27]
    xe_ref, xd_ref, cnt_ref, cls_ref = refs[27:]

    m = _NS * _TPAD
    row2 = lax.broadcasted_iota(jnp.int32, (m, m), 0)
    col2 = lax.broadcasted_iota(jnp.int32, (m, m), 1)
    same = (row2 // _TPAD) == (col2 // _TPAD)
    r_t = row2 % _TPAD
    c_t = col2 % _TPAD
    rows1 = lax.broadcasted_iota(jnp.int32, (m, 1), 0) % _TPAD
    sel_cache = {}

    def sel(stride, off):
        key = (stride, off)
        if key not in sel_cache:
            tgt = r_t * stride + off
            hit = jnp.logical_and(tgt >= 0, c_t == tgt)
            sel_cache[key] = jnp.where(jnp.logical_and(same, hit),
                                       1.0, 0.0).astype(jnp.bfloat16)
        return sel_cache[key]

    h = x_ref[...].astype(jnp.float32)
    resid = None
    for ci, (taps, relu, tv) in enumerate(_ENC_CONVS):
        for start, _ in _ENC_RES:
            if ci == start:
                resid = h
                h = jnp.maximum(h, 0.0)
        w3d = wr[2 * ci][...]
        b = wr[2 * ci + 1][...]
        h_bf = h.astype(jnp.bfloat16)
        acc = None
        for q, (stride, off) in enumerate(taps):
            if stride == 0:
                src = h_bf
            else:
                src = jnp.dot(sel(stride, off), h_bf,
                              preferred_element_type=jnp.float32
                              ).astype(jnp.bfloat16)
            y = jnp.dot(src, w3d[q], preferred_element_type=jnp.float32)
            acc = y if acc is None else acc + y
        acc = acc + b
        if relu:
            acc = jnp.maximum(acc, 0.0)
        if tv < _TPAD:
            acc = jnp.where(rows1 < tv, acc, 0.0)
        h = acc
        for _, end in _ENC_RES:
            if ci == end:
                h = h + resid

    cb = cb_ref[...]
    ncp = cb.shape[0]
    x_sq = jnp.sum(h * h, axis=-1, keepdims=True)
    xc = lax.dot_general(h, cb, (((1,), (1,)), ((), ())),
                         preferred_element_type=jnp.float32)
    dist = x_sq - 2.0 * xc + cbsq_ref[...]
    col = lax.broadcasted_iota(jnp.int32, (m, ncp), 1)
    min_d = jnp.min(dist, axis=-1, keepdims=True)
    best = jnp.min(jnp.where(dist <= min_d, col, jnp.int32(2 ** 30)),
                   axis=-1, keepdims=True)
    onehot = jnp.where(jnp.logical_and(col == best, rows1 < 128), 1.0, 0.0)
    xd = jnp.dot(onehot, cb, preferred_element_type=jnp.float32)

    xe_ref[...] = jnp.concatenate([h[0:8], h[_TPAD:_TPAD + 8]], axis=0)
    xd_ref[...] = jnp.concatenate([xd[0:128], xd[_TPAD:_TPAD + 128]], axis=0)
    ind8 = (lax.broadcasted_iota(jnp.int32, (8, 1), 0) == 0).astype(jnp.float32)
    cnt_ref[...] = ind8 * jnp.sum(onehot, axis=0, keepdims=True)
    diff = h - xd
    cls_ref[...] = ind8 * jnp.sum(diff * diff, axis=0, keepdims=True)


def _shift(x, off):
    if off == 0:
        return x
    t = x.shape[0]
    z = jnp.zeros((abs(off), x.shape[1]), x.dtype)
    if off > 0:
        return jnp.concatenate([x[off:, :], z], axis=0)
    return jnp.concatenate([z, x[: t + off, :]], axis=0)


def _taps(srcs, w, b):
    ci = srcs[0].shape[1]
    acc = None
    for q, s in enumerate(srcs):
        y = jnp.dot(s, w[q * ci:(q + 1) * ci, :],
                    preferred_element_type=jnp.float32)
        acc = y if acc is None else acc + y
    return acc + b


def _conv(h, offs, w, b, relu):
    hb = h.astype(jnp.bfloat16)
    y = _taps([_shift(hb, o) for o in offs], w, b)
    if relu:
        y = jnp.maximum(y, 0.0)
    return y


def _res(h, dil, wa, ba, wb, bb):
    t = jnp.maximum(h, 0.0)
    t = _conv(t, (-dil, 0, dil), wa, ba, True)
    t = _conv(t, (0,), wb, bb, False)
    return h + t


def _up(h, qe_full, w, b):
    t = h.shape[0]
    qe = qe_full[: 2 * t, :t]
    hb = h.astype(jnp.bfloat16)
    ev = _taps([_shift(hb, -1), hb, hb], w, b)
    od = _taps([hb, hb, _shift(hb, 1)], w, b)
    u = jnp.dot(qe, ev, preferred_element_type=jnp.float32)
    return u + _shift(jnp.dot(qe, od, preferred_element_type=jnp.float32), -1)


def _dec_kernel(*refs):
    xd_ref, qe_ref = refs[:2]
    dw = [r[...] for r in refs[2:-1]]
    o_ref = refs[-1]
    qe = qe_ref[...]
    xdv = xd_ref[...]
    for n in range(_NS):
        h = xdv[n * 128:(n + 1) * 128, :]
        h = _conv(h, (-1, 0, 1), dw[0], dw[1], True)
        h = _res(h, 3, dw[2], dw[3], dw[4], dw[5])
        h = _res(h, 1, dw[6], dw[7], dw[8], dw[9])
        h = _up(h, qe, dw[10], dw[11])
        h = _res(h, 3, dw[12], dw[13], dw[14], dw[15])
        h = _res(h, 1, dw[16], dw[17], dw[18], dw[19])
        h = _up(h, qe, dw[20], dw[21])
        h = _conv(h, (-1, 0, 1), dw[22], dw[23], True)
        h = _conv(h, (-1, 0, 1), dw[24], dw[25], False)
        o_ref[n * _TPAD:(n + 1) * _TPAD, :] = h


def _sim_kernel(te_ref, tw_ref, tb_ref, zr_ref, zj_ref, pos_ref, neg_ref,
                it_ref, o_ref):
    tp = jnp.dot(te_ref[...], tw_ref[...],
                 preferred_element_type=jnp.float32) + tb_ref[...]
    zi = jnp.concatenate([tp, zr_ref[...]], axis=0)
    zj = zj_ref[...]
    zi = zi * lax.rsqrt(jnp.sum(zi * zi, axis=1, keepdims=True) + 1e-12)
    zj = zj * lax.rsqrt(jnp.sum(zj * zj, axis=1, keepdims=True) + 1e-12)
    s = lax.dot_general(zj, zi, (((1,), (1,)), ((), ())),
                        preferred_element_type=jnp.float32) * it_ref[...]
    e = jnp.exp(s - jnp.max(s, axis=1, keepdims=True))
    p = pos_ref[...]
    sp = jnp.sum(e * p, axis=1, keepdims=True)
    sn = jnp.sum(e * neg_ref[...], axis=1, keepdims=True)
    lp = jnp.where(sp > 0.0, jnp.log(sp / jnp.maximum(sn, 1e-30)), 0.0)
    o_ref[...] = (-jnp.sum(p, axis=1, keepdims=True) * lp) * jnp.ones(
        (1, 128), jnp.float32)


def _round_up8(x):
    return (x + 7) // 8 * 8


def kernel(gesture, audio_embeddings, text_embeddings, vid_indices, cb, cbsq,
           text_w, text_b,
           enc_w0, enc_b0, enc_w1, enc_b1, enc_w2, enc_b2, enc_w3, enc_b3,
           enc_w4, enc_b4, enc_w5, enc_b5, enc_w6, enc_b6, enc_w7, enc_b7,
           enc_w8, enc_b8, enc_w9, enc_b9, enc_w10, enc_b10, enc_w11, enc_b11,
           dec_w0, dec_b0, dec_w1, dec_b1, dec_w2, dec_b2, dec_w3, dec_b3,
           dec_w4, dec_b4, dec_w5, dec_b5, dec_w6, dec_b6, dec_w7, dec_b7,
           dec_w8, dec_b8, dec_w9, dec_b9, dec_w10, dec_b10, dec_w11, dec_b11,
           dec_w12, dec_b12):
    B, T, cin = gesture.shape
    D = cb.shape[1]
    ncp = cb.shape[0]
    tc = T // 4
    out_pose = 48
    temperature = 0.1
    nblk = B // _NS

    x2d = gesture.astype(jnp.float32).reshape(B * T, cin)

    eargs = [x2d]
    in_specs = [pl.BlockSpec((_NS * T, cin), lambda i: (i, 0))]
    for w, b in ((enc_w0, enc_b0), (enc_w1, enc_b1), (enc_w2, enc_b2),
                 (enc_w3, enc_b3), (enc_w4, enc_b4), (enc_w5, enc_b5),
                 (enc_w6, enc_b6), (enc_w7, enc_b7), (enc_w8, enc_b8),
                 (enc_w9, enc_b9), (enc_w10, enc_b10), (enc_w11, enc_b11)):
        eargs += [w, b]
        in_specs += [pl.BlockSpec(w.shape, lambda i: (0, 0, 0)),
                     pl.BlockSpec(b.shape, lambda i: (0, 0))]
    eargs += [cb, cbsq]
    in_specs += [pl.BlockSpec(cb.shape, lambda i: (0, 0)),
                 pl.BlockSpec(cbsq.shape, lambda i: (0, 0))]

    xe_head, xd_c, cnt, cls = pl.pallas_call(
        _enc_vq_kernel,
        out_shape=(jax.ShapeDtypeStruct((nblk * 16, D), jnp.float32),
                   jax.ShapeDtypeStruct((B * tc, D), jnp.float32),
                   jax.ShapeDtypeStruct((nblk * 8, ncp), jnp.float32),
                   jax.ShapeDtypeStruct((nblk * 8, D), jnp.float32)),
        grid=(nblk,),
        in_specs=in_specs,
        out_specs=(pl.BlockSpec((16, D), lambda i: (i, 0)),
                   pl.BlockSpec((_NS * tc, D), lambda i: (i, 0)),
                   pl.BlockSpec((8, ncp), lambda i: (i, 0)),
                   pl.BlockSpec((8, D), lambda i: (i, 0))),
        compiler_params=pltpu.CompilerParams(
            dimension_semantics=("parallel",),
            vmem_limit_bytes=_VMEM_LIMIT),
    )(*eargs)

    def cat(w):
        k, ci, co = w.shape
        return w.reshape(k * ci, co)

    qe = (jnp.arange(T)[:, None] == 2 * jnp.arange(T // 2)[None, :]
          ).astype(jnp.float32)
    dargs = [xd_c, qe]
    for w, b in ((dec_w0, dec_b0), (dec_w1, dec_b1), (dec_w2, dec_b2),
                 (dec_w3, dec_b3), (dec_w4, dec_b4), (dec_w5, dec_b5),
                 (dec_w6, dec_b6), (dec_w7, dec_b7), (dec_w8, dec_b8),
                 (dec_w9, dec_b9), (dec_w10, dec_b10), (dec_w11, dec_b11),
                 (dec_w12, dec_b12)):
        dargs += [cat(w), b]
    d_specs = [pl.BlockSpec((_NS * tc, D), lambda i: (i, 0))]
    for a in dargs[1:]:
        d_specs.append(pl.BlockSpec(a.shape, lambda i: (0, 0)))

    dec = pl.pallas_call(
        _dec_kernel,
        out_shape=jax.ShapeDtypeStruct((B * T, 128), jnp.float32),
        grid=(nblk,),
        in_specs=d_specs,
        out_specs=pl.BlockSpec((_NS * T, 128), lambda i: (i, 0)),
        compiler_params=pltpu.CompilerParams(
            dimension_semantics=("parallel",),
            vmem_limit_bytes=_VMEM_LIMIT),
    )(*dargs)
    x_out = dec.reshape(B, T, 128)[:, :, :out_pose]

    m_valid = B * tc
    counts = jnp.sum(cnt, axis=0)[:ncp]
    prob = counts / m_valid
    perplexity = jnp.exp(-jnp.sum(prob * jnp.log(prob + 1e-7)))
    loss_commit = jnp.sum(cls) / (m_valid * D)

    xe3 = xe_head.reshape(B, 8, D)
    g1, g2, g3, g4 = (xe3[:, t, :] for t in range(4))
    audio = audio_embeddings.reshape(-1, 512).astype(jnp.float32)
    text = text_embeddings.reshape(-1, 768).astype(jnp.float32)

    n = text.shape[0]
    npad = _round_up8(max(n, 8))

    def padr(z):
        return jnp.pad(z, ((0, npad - z.shape[0]), (0, 0)))

    def padm(mm):
        return jnp.pad(mm, ((0, npad - n), (0, npad - n)))

    zj = jnp.concatenate([padr(g1), padr(g2), padr(g3), padr(g4)], axis=0)
    zrest = jnp.concatenate([padr(audio), padr(g3), padr(g4)], axis=0)

    eye = jnp.eye(n, dtype=jnp.float32)
    cats = jnp.asarray(vid_indices, jnp.int32)
    style = (cats[None, :] == cats[:, None]).astype(jnp.float32)
    ones = jnp.ones((n, n), jnp.float32)
    R = 4 * npad
    pos = jnp.zeros((R, R), jnp.float32)
    neg = jnp.zeros((R, R), jnp.float32)
    for k, (pm, nm) in enumerate([(eye, ones - eye), (eye, ones - eye),
                                  (style, ones - style), (style, ones - style)]):
        o = k * npad
        pos = pos.at[o:o + npad, o:o + npad].set(padm(pm))
        neg = neg.at[o:o + npad, o:o + npad].set(padm(nm))
    invt = jnp.concatenate(
        [jnp.full((2 * npad, 1), 1.0 / temperature, jnp.float32),
         jnp.ones((2 * npad, 1), jnp.float32)], axis=0)

    sim_args = [padr(text), text_w, text_b, zrest, zj, pos, neg, invt]
    sim_out = pl.pallas_call(
        _sim_kernel,
        out_shape=jax.ShapeDtypeStruct((R, 128), jnp.float32),
        grid=(1,),
        in_specs=[pl.BlockSpec(a.shape, lambda i, nd=a.ndim: (0,) * nd)
                  for a in sim_args],
        out_specs=pl.BlockSpec((R, 128), lambda i: (0, 0)),
        compiler_params=pltpu.CompilerParams(vmem_limit_bytes=_VMEM_LIMIT),
    )(*sim_args)

    case = sim_out[:, 0].reshape(4, npad).sum(axis=1) / n
    gesture_text_loss = case[0]
    gesture_audio_loss = case[1]
    gesture_style_loss = (case[2] + case[3]) / 2.0

    return (x_out, loss_commit, perplexity, gesture_text_loss,
            gesture_audio_loss, gesture_style_loss)


# speedup vs baseline: 1.6553x; 1.0020x over previous
import functools

import jax
import jax.numpy as jnp
from jax import lax
from jax.experimental import pallas as pl
from jax.experimental.pallas import tpu as pltpu


_VMEM_LIMIT = 48 * 1024 * 1024
_TPAD = 512
_NS = 2

_ENC_CONVS = (
    (((1, -1), (0, 0), (1, 1)), True, 512),
    (((2, -1), (2, 0), (2, 1), (2, 2)), False, 256),
    (((1, -1), (0, 0), (1, 1)), True, 256),
    (((0, 0),), False, 256),
    (((1, -3), (0, 0), (1, 3)), True, 256),
    (((0, 0),), False, 256),
    (((2, -1), (2, 0), (2, 1), (2, 2)), False, 128),
    (((1, -1), (0, 0), (1, 1)), True, 128),
    (((0, 0),), False, 128),
    (((1, -3), (0, 0), (1, 3)), True, 128),
    (((0, 0),), False, 128),
    (((1, -1), (0, 0), (1, 1)), False, 128),
)
_ENC_RES = ((2, 3), (4, 5), (7, 8), (9, 10))


def _enc_vq_kernel(*refs):
    x_ref = refs[0]
    wr = refs[1:25]
    cb_ref, cbsq_ref = refs[25:27]
    xe_ref, xd_ref, cnt_ref, cls_ref = refs[27:]

    m = _NS * _TPAD
    row2 = lax.broadcasted_iota(jnp.int32, (m, m), 0)
    col2 = lax.broadcasted_iota(jnp.int32, (m, m), 1)
    same = (row2 // _TPAD) == (col2 // _TPAD)
    r_t = row2 % _TPAD
    c_t = col2 % _TPAD
    rows1 = lax.broadcasted_iota(jnp.int32, (m, 1), 0) % _TPAD
    sel_cache = {}

    def sel(stride, off):
        key = (stride, off)
        if key not in sel_cache:
            tgt = r_t * stride + off
            hit = jnp.logical_and(tgt >= 0, c_t == tgt)
            sel_cache[key] = jnp.where(jnp.logical_and(same, hit),
                                       1.0, 0.0).astype(jnp.bfloat16)
        return sel_cache[key]

    h = x_ref[...].astype(jnp.float32)
    resid = None
    for ci, (taps, relu, tv) in enumerate(_ENC_CONVS):
        for start, _ in _ENC_RES:
            if ci == start:
                resid = h
                h = jnp.maximum(h, 0.0)
        w3d = wr[2 * ci][...]
        b = wr[2 * ci + 1][...]
        h_bf = h.astype(jnp.bfloat16)
        acc = None
        for q, (stride, off) in enumerate(taps):
            if stride == 0:
                src = h_bf
            else:
                src = jnp.dot(sel(stride, off), h_bf,
                              preferred_element_type=jnp.float32
                              ).astype(jnp.bfloat16)
            y = jnp.dot(src, w3d[q], preferred_element_type=jnp.float32)
            acc = y if acc is None else acc + y
        acc = acc + b
        if relu:
            acc = jnp.maximum(acc, 0.0)
        if tv < _TPAD:
            acc = jnp.where(rows1 < tv, acc, 0.0)
        h = acc
        for _, end in _ENC_RES:
            if ci == end:
                h = h + resid

    cb = cb_ref[...]
    ncp = cb.shape[0]
    x_sq = jnp.sum(h * h, axis=-1, keepdims=True)
    xc = lax.dot_general(h, cb, (((1,), (1,)), ((), ())),
                         preferred_element_type=jnp.float32)
    dist = x_sq - 2.0 * xc + cbsq_ref[...]
    col = lax.broadcasted_iota(jnp.int32, (m, ncp), 1)
    min_d = jnp.min(dist, axis=-1, keepdims=True)
    best = jnp.min(jnp.where(dist <= min_d, col, jnp.int32(2 ** 30)),
                   axis=-1, keepdims=True)
    onehot = jnp.where(jnp.logical_and(col == best, rows1 < 128), 1.0, 0.0)
    xd = jnp.dot(onehot, cb, preferred_element_type=jnp.float32)

    xe_ref[...] = jnp.concatenate([h[0:8], h[_TPAD:_TPAD + 8]], axis=0)
    xd_ref[...] = jnp.concatenate([xd[0:128], xd[_TPAD:_TPAD + 128]], axis=0)
    ind8 = (lax.broadcasted_iota(jnp.int32, (8, 1), 0) == 0).astype(jnp.float32)
    cnt_ref[...] = ind8 * jnp.sum(onehot, axis=0, keepdims=True)
    diff = h - xd
    cls_ref[...] = ind8 * jnp.sum(diff * diff, axis=0, keepdims=True)


def _shift(x, off):
    if off == 0:
        return x
    t = x.shape[0]
    z = jnp.zeros((abs(off), x.shape[1]), x.dtype)
    if off > 0:
        return jnp.concatenate([x[off:, :], z], axis=0)
    return jnp.concatenate([z, x[: t + off, :]], axis=0)


def _taps(srcs, w, b):
    ci = srcs[0].shape[1]
    acc = None
    for q, s in enumerate(srcs):
        y = jnp.dot(s, w[q * ci:(q + 1) * ci, :],
                    preferred_element_type=jnp.float32)
        acc = y if acc is None else acc + y
    return acc + b


def _conv(h, offs, w, b, relu):
    hb = h.astype(jnp.bfloat16)
    y = _taps([_shift(hb, o) for o in offs], w, b)
    if relu:
        y = jnp.maximum(y, 0.0)
    return y


def _res(h, dil, wa, ba, wb, bb):
    t = jnp.maximum(h, 0.0)
    t = _conv(t, (-dil, 0, dil), wa, ba, True)
    t = _conv(t, (0,), wb, bb, False)
    return h + t


def _up(h, qe_full, w, b):
    t = h.shape[0]
    qe = qe_full[: 2 * t, :t]
    hb = h.astype(jnp.bfloat16)
    ev = _taps([_shift(hb, -1), hb, hb], w, b)
    od = _taps([hb, hb, _shift(hb, 1)], w, b)
    u = jnp.dot(qe, ev, preferred_element_type=jnp.float32)
    return u + _shift(jnp.dot(qe, od, preferred_element_type=jnp.float32), -1)


def _dec_kernel(*refs):
    xd_ref, qe_ref = refs[:2]
    dw = [r[...] for r in refs[2:-1]]
    o_ref = refs[-1]
    qe = qe_ref[...]
    xdv = xd_ref[...]
    for n in range(_NS):
        h = xdv[n * 128:(n + 1) * 128, :]
        h = _conv(h, (-1, 0, 1), dw[0], dw[1], True)
        h = _res(h, 3, dw[2], dw[3], dw[4], dw[5])
        h = _res(h, 1, dw[6], dw[7], dw[8], dw[9])
        h = _up(h, qe, dw[10], dw[11])
        h = _res(h, 3, dw[12], dw[13], dw[14], dw[15])
        h = _res(h, 1, dw[16], dw[17], dw[18], dw[19])
        h = _up(h, qe, dw[20], dw[21])
        h = _conv(h, (-1, 0, 1), dw[22], dw[23], True)
        h = _conv(h, (-1, 0, 1), dw[24], dw[25], False)
        o_ref[n * _TPAD:(n + 1) * _TPAD, :] = h


def _sim_kernel(te_ref, tw_ref, tb_ref, zr_ref, zj_ref, vr_ref, vc_ref,
                o_ref, *, npad, inv_t):
    tp = jnp.dot(te_ref[...], tw_ref[...],
                 preferred_element_type=jnp.float32) + tb_ref[...]
    zi = jnp.concatenate([tp, zr_ref[...]], axis=0)
    zj = zj_ref[...]
    zi = zi * lax.rsqrt(jnp.sum(zi * zi, axis=1, keepdims=True) + 1e-12)
    zj = zj * lax.rsqrt(jnp.sum(zj * zj, axis=1, keepdims=True) + 1e-12)
    r = zj.shape[0]
    row1 = lax.broadcasted_iota(jnp.int32, (r, 1), 0)
    invt = jnp.where(row1 < 2 * npad, jnp.float32(inv_t), 1.0)
    s = lax.dot_general(zj, zi, (((1,), (1,)), ((), ())),
                        preferred_element_type=jnp.float32) * invt
    row2 = lax.broadcasted_iota(jnp.int32, (r, r), 0)
    col2 = lax.broadcasted_iota(jnp.int32, (r, r), 1)
    same_case = jnp.where((row2 // npad) == (col2 // npad), 1.0, 0.0)
    diag = jnp.where((row2 % npad) == (col2 % npad), 1.0, 0.0)
    style = jnp.where(vc_ref[...] == vr_ref[...], 1.0, 0.0)
    iseye = jnp.where(row2 < 2 * npad, 1.0, 0.0)
    hit = iseye * diag + (1.0 - iseye) * style
    valid = (jnp.where(vr_ref[...] >= 0.0, 1.0, 0.0)
             * jnp.where(vc_ref[...] >= 0.0, 1.0, 0.0))
    same_case = same_case * valid
    p = same_case * hit
    ng = same_case * (1.0 - hit)
    e = jnp.exp(s - jnp.max(s, axis=1, keepdims=True))
    sp = jnp.sum(e * p, axis=1, keepdims=True)
    sn = jnp.sum(e * ng, axis=1, keepdims=True)
    lp = jnp.where(sp > 0.0, jnp.log(sp / jnp.maximum(sn, 1e-30)), 0.0)
    loss = -jnp.sum(p, axis=1, keepdims=True) * lp
    lane = lax.broadcasted_iota(jnp.int32, (1, 128), 1)
    acc = jnp.zeros((1, 128), jnp.float32)
    for c in range(4):
        sc = jnp.sum(jnp.where(row1 // npad == c, loss, 0.0))
        acc = acc + jnp.where(lane == c, sc, 0.0)
    rows8 = lax.broadcasted_iota(jnp.int32, (8, 1), 0)
    o_ref[...] = jnp.where(rows8 == 0, acc, 0.0)


def _round_up8(x):
    return (x + 7) // 8 * 8


def kernel(gesture, audio_embeddings, text_embeddings, vid_indices, cb, cbsq,
           text_w, text_b,
           enc_w0, enc_b0, enc_w1, enc_b1, enc_w2, enc_b2, enc_w3, enc_b3,
           enc_w4, enc_b4, enc_w5, enc_b5, enc_w6, enc_b6, enc_w7, enc_b7,
           enc_w8, enc_b8, enc_w9, enc_b9, enc_w10, enc_b10, enc_w11, enc_b11,
           dec_w0, dec_b0, dec_w1, dec_b1, dec_w2, dec_b2, dec_w3, dec_b3,
           dec_w4, dec_b4, dec_w5, dec_b5, dec_w6, dec_b6, dec_w7, dec_b7,
           dec_w8, dec_b8, dec_w9, dec_b9, dec_w10, dec_b10, dec_w11, dec_b11,
           dec_w12, dec_b12):
    B, T, cin = gesture.shape
    D = cb.shape[1]
    ncp = cb.shape[0]
    tc = T // 4
    out_pose = 48
    temperature = 0.1
    nblk = B // _NS

    x2d = gesture.astype(jnp.float32).reshape(B * T, cin)

    eargs = [x2d]
    in_specs = [pl.BlockSpec((_NS * T, cin), lambda i: (i, 0))]
    for w, b in ((enc_w0, enc_b0), (enc_w1, enc_b1), (enc_w2, enc_b2),
                 (enc_w3, enc_b3), (enc_w4, enc_b4), (enc_w5, enc_b5),
                 (enc_w6, enc_b6), (enc_w7, enc_b7), (enc_w8, enc_b8),
                 (enc_w9, enc_b9), (enc_w10, enc_b10), (enc_w11, enc_b11)):
        eargs += [w, b]
        in_specs += [pl.BlockSpec(w.shape, lambda i: (0, 0, 0)),
                     pl.BlockSpec(b.shape, lambda i: (0, 0))]
    eargs += [cb, cbsq]
    in_specs += [pl.BlockSpec(cb.shape, lambda i: (0, 0)),
                 pl.BlockSpec(cbsq.shape, lambda i: (0, 0))]

    xe_head, xd_c, cnt, cls = pl.pallas_call(
        _enc_vq_kernel,
        out_shape=(jax.ShapeDtypeStruct((nblk * 16, D), jnp.float32),
                   jax.ShapeDtypeStruct((B * tc, D), jnp.float32),
                   jax.ShapeDtypeStruct((nblk * 8, ncp), jnp.float32),
                   jax.ShapeDtypeStruct((nblk * 8, D), jnp.float32)),
        grid=(nblk,),
        in_specs=in_specs,
        out_specs=(pl.BlockSpec((16, D), lambda i: (i, 0)),
                   pl.BlockSpec((_NS * tc, D), lambda i: (i, 0)),
                   pl.BlockSpec((8, ncp), lambda i: (i, 0)),
                   pl.BlockSpec((8, D), lambda i: (i, 0))),
        compiler_params=pltpu.CompilerParams(
            dimension_semantics=("parallel",),
            vmem_limit_bytes=_VMEM_LIMIT),
    )(*eargs)

    def cat(w):
        k, ci, co = w.shape
        return w.reshape(k * ci, co)

    qe = (jnp.arange(T)[:, None] == 2 * jnp.arange(T // 2)[None, :]
          ).astype(jnp.float32)
    dargs = [xd_c, qe]
    for w, b in ((dec_w0, dec_b0), (dec_w1, dec_b1), (dec_w2, dec_b2),
                 (dec_w3, dec_b3), (dec_w4, dec_b4), (dec_w5, dec_b5),
                 (dec_w6, dec_b6), (dec_w7, dec_b7), (dec_w8, dec_b8),
                 (dec_w9, dec_b9), (dec_w10, dec_b10), (dec_w11, dec_b11),
                 (dec_w12, dec_b12)):
        dargs += [cat(w), b]
    dargs[-2] = dargs[-2][:, :out_pose]
    dargs[-1] = dargs[-1][:, :out_pose]
    d_specs = [pl.BlockSpec((_NS * tc, D), lambda i: (i, 0))]
    for a in dargs[1:]:
        d_specs.append(pl.BlockSpec(a.shape, lambda i: (0, 0)))

    dec = pl.pallas_call(
        _dec_kernel,
        out_shape=jax.ShapeDtypeStruct((B * T, out_pose), jnp.float32),
        grid=(nblk,),
        in_specs=d_specs,
        out_specs=pl.BlockSpec((_NS * T, out_pose), lambda i: (i, 0)),
        compiler_params=pltpu.CompilerParams(
            dimension_semantics=("parallel",),
            vmem_limit_bytes=_VMEM_LIMIT),
    )(*dargs)
    x_out = dec.reshape(B, T, out_pose)

    m_valid = B * tc
    counts = jnp.sum(cnt, axis=0)[:ncp]
    prob = counts / m_valid
    perplexity = jnp.exp(-jnp.sum(prob * jnp.log(prob + 1e-7)))
    loss_commit = jnp.sum(cls) / (m_valid * D)

    xe3 = xe_head.reshape(B, 8, D)
    g1, g2, g3, g4 = (xe3[:, t, :] for t in range(4))
    audio = audio_embeddings.reshape(-1, 512).astype(jnp.float32)
    text = text_embeddings.reshape(-1, 768).astype(jnp.float32)

    n = text.shape[0]
    npad = _round_up8(max(n, 8))

    def padr(z):
        return jnp.pad(z, ((0, npad - z.shape[0]), (0, 0)))

    zj = jnp.concatenate([padr(g1), padr(g2), padr(g3), padr(g4)], axis=0)
    zrest = jnp.concatenate([padr(audio), padr(g3), padr(g4)], axis=0)

    R = 4 * npad
    cats = jnp.asarray(vid_indices, jnp.float32)
    vpad = jnp.pad(cats, (0, npad - n), constant_values=-1.0)
    v4 = jnp.tile(vpad, 4)
    sim_args = [padr(text), text_w, text_b, zrest, zj,
                v4.reshape(1, R), v4.reshape(R, 1)]
    sim_out = pl.pallas_call(
        functools.partial(_sim_kernel, npad=npad, inv_t=1.0 / temperature),
        out_shape=jax.ShapeDtypeStruct((8, 128), jnp.float32),
        grid=(1,),
        in_specs=[pl.BlockSpec(a.shape, lambda i, nd=a.ndim: (0,) * nd)
                  for a in sim_args],
        out_specs=pl.BlockSpec((8, 128), lambda i: (0, 0)),
        compiler_params=pltpu.CompilerParams(vmem_limit_bytes=_VMEM_LIMIT),
    )(*sim_args)

    case = sim_out[0, :4] / n
    gesture_text_loss = case[0]
    gesture_audio_loss = case[1]
    gesture_style_loss = (case[2] + case[3]) / 2.0

    return (x_out, loss_commit, perplexity, gesture_text_loss,
            gesture_audio_loss, gesture_style_loss)
